# baseline (speedup 1.0000x reference)
import jax
import jax.numpy as jnp
from jax import lax
from jax.experimental import pallas as pl
from jax.experimental.pallas import tpu as pltpu
from jax.experimental.pallas import tpu_sc as plsc

BATCH = 16384
DIM = 64
VOCAB = 1000000
NC = 2
NS = 16
NW = NC * NS
BPW = BATCH // NW
L = 16
BPC = 64
NCHUNK = BPW // BPC
GPC = BPC // L
CHE = BPC * DIM


def _body(tw_hbm, cw_hbm, tflat_hbm, cflat_hbm, out_hbm,
          idx_t, idx_c, ei_t0, ei_t1, ei_c0, ei_c1,
          dt0, dt1, dc0, dc1, out_v,
          st0, st1, sc0, sc1):
    ei_t = (ei_t0, ei_t1)
    ei_c = (ei_c0, ei_c1)
    dt = (dt0, dt1)
    dc = (dc0, dc1)
    cid = lax.axis_index("c")
    sid = lax.axis_index("s")
    wid = sid * NC + cid
    base = wid * BPW

    for j in range(NCHUNK):
        pltpu.sync_copy(tw_hbm.at[pl.ds(base + j * BPC, BPC)], idx_t.at[j])
        pltpu.sync_copy(cw_hbm.at[pl.ds(base + j * BPC, BPC)], idx_c.at[j])

    def build(slot, ch):
        for g in range(GPC):
            vt = idx_t[ch, pl.ds(g * L, L)]
            vc = idx_c[ch, pl.ds(g * L, L)]
            for d in range(DIM):
                o = (g * DIM + d) * L
                ei_t[slot][pl.ds(o, L)] = vt + d * VOCAB
                ei_c[slot][pl.ds(o, L)] = vc + d * VOCAB

    def copies(slot):
        semt = st0 if slot == 0 else st1
        semc = sc0 if slot == 0 else sc1
        return (
            pltpu.make_async_copy(tflat_hbm.at[ei_t[slot]], dt[slot], semt),
            pltpu.make_async_copy(cflat_hbm.at[ei_c[slot]], dc[slot], semc),
        )

    def start(slot):
        for c in copies(slot):
            c.start()

    def wait(slot):
        for c in copies(slot):
            c.wait()

    def compute(slot, ch):
        for g in range(GPC):
            acc = jnp.zeros((L,), jnp.float32)
            for d in range(DIM):
                o = (g * DIM + d) * L
                acc = acc + dt[slot][pl.ds(o, L)] * dc[slot][pl.ds(o, L)]
            out_v[pl.ds(ch * BPC + g * L, L)] = acc

    build(0, 0)
    start(0)
    build(1, 1)
    start(1)

    def step(s, carry):
        ch0 = 2 * s
        ch1 = 2 * s + 1
        wait(0)
        compute(0, ch0)
        @pl.when(ch0 + 2 < NCHUNK)
        def _():
            build(0, ch0 + 2)
            start(0)
        wait(1)
        compute(1, ch1)
        @pl.when(ch1 + 2 < NCHUNK)
        def _():
            build(1, ch1 + 2)
            start(1)
        return carry

    lax.fori_loop(0, NCHUNK // 2, step, 0)

    pltpu.sync_copy(out_v, out_hbm.at[pl.ds(base, BPW)])


def kernel(target_word, context_word, target_emb, context_emb):
    tw = target_word.astype(jnp.int32)
    cw = context_word.astype(jnp.int32)
    tflat = target_emb.T.reshape(-1)
    cflat = context_emb.T.reshape(-1)
    mesh = plsc.VectorSubcoreMesh(
        core_axis_name="c", subcore_axis_name="s",
        num_cores=NC, num_subcores=NS)
    run = pl.kernel(
        _body,
        out_type=jax.ShapeDtypeStruct((BATCH,), jnp.float32),
        mesh=mesh,
        scratch_types=[
            pltpu.VMEM((NCHUNK, BPC), jnp.int32),
            pltpu.VMEM((NCHUNK, BPC), jnp.int32),
            pltpu.VMEM((CHE,), jnp.int32),
            pltpu.VMEM((CHE,), jnp.int32),
            pltpu.VMEM((CHE,), jnp.int32),
            pltpu.VMEM((CHE,), jnp.int32),
            pltpu.VMEM((CHE,), jnp.float32),
            pltpu.VMEM((CHE,), jnp.float32),
            pltpu.VMEM((CHE,), jnp.float32),
            pltpu.VMEM((CHE,), jnp.float32),
            pltpu.VMEM((BPW,), jnp.float32),
            pltpu.SemaphoreType.DMA,
            pltpu.SemaphoreType.DMA,
            pltpu.SemaphoreType.DMA,
            pltpu.SemaphoreType.DMA,
        ],
        compiler_params=pltpu.CompilerParams(needs_layout_passes=False),
    )
    return run(tw, cw, tflat, cflat)


# speedup vs baseline: 7.1974x; 7.1974x over previous
import jax
import jax.numpy as jnp
from jax import lax
from jax.experimental import pallas as pl
from jax.experimental.pallas import tpu as pltpu
from jax.experimental.pallas import tpu_sc as plsc

BATCH = 16384
DIM = 64
VOCAB = 1000000
LINES = VOCAB // 2
LINE_W = 128
NC = 2
NS = 16
NW = NC * NS
BPW = BATCH // NW
L = 16
CH = 128
NCHUNK = BPW // CH

W = 512
MAIN = 999936
TGRID = MAIN // W
HLINES = W // 2
TAIL = VOCAB - MAIN
TAIL_LINE0 = MAIN // 2


def _transpose_body(ta_hbm, ca_hbm, tt_ref, tc_ref, to_hbm, co_hbm,
                    buf, ot, oc, isem, osem):
    i = pl.program_id(0)

    def start_in(slot, step):
        pltpu.make_async_copy(
            ta_hbm.at[:, pl.ds(step * W, W)], buf.at[slot, 0],
            isem.at[slot, 0]).start()
        pltpu.make_async_copy(
            ca_hbm.at[:, pl.ds(step * W, W)], buf.at[slot, 1],
            isem.at[slot, 1]).start()

    def wait_in(slot):
        for k in range(2):
            pltpu.make_async_copy(
                ta_hbm.at[:, pl.ds(0, W)], buf.at[slot, k], isem.at[slot, k]
            ).wait()

    @pl.when(i == 0)
    def _():
        start_in(0, 0)
    slot = lax.rem(i, 2)
    @pl.when(i + 1 < TGRID)
    def _():
        start_in(1 - slot, i + 1)
    @pl.when(i >= 2)
    def _():
        pltpu.make_async_copy(
            ot.at[slot], to_hbm.at[pl.ds(0, HLINES)], osem.at[slot, 0]).wait()
        pltpu.make_async_copy(
            oc.at[slot], co_hbm.at[pl.ds(0, HLINES)], osem.at[slot, 1]).wait()
    wait_in(slot)
    ot[slot, :, 0:DIM] = buf[slot, 0, :, 0:HLINES].T
    ot[slot, :, DIM:LINE_W] = buf[slot, 0, :, HLINES:W].T
    oc[slot, :, 0:DIM] = buf[slot, 1, :, 0:HLINES].T
    oc[slot, :, DIM:LINE_W] = buf[slot, 1, :, HLINES:W].T
    pltpu.make_async_copy(
        ot.at[slot], to_hbm.at[pl.ds(i * HLINES, HLINES)],
        osem.at[slot, 0]).start()
    pltpu.make_async_copy(
        oc.at[slot], co_hbm.at[pl.ds(i * HLINES, HLINES)],
        osem.at[slot, 1]).start()

    @pl.when(i == TGRID - 1)
    def _():
        pltpu.make_async_copy(
            ot.at[slot], to_hbm.at[pl.ds(0, HLINES)], osem.at[slot, 0]).wait()
        pltpu.make_async_copy(
            oc.at[slot], co_hbm.at[pl.ds(0, HLINES)], osem.at[slot, 1]).wait()
        tl = jnp.concatenate([tt_ref[0:32, :], tt_ref[32:64, :]], axis=1)
        cl = jnp.concatenate([tc_ref[0:32, :], tc_ref[32:64, :]], axis=1)
        ot[slot, 0:32, :] = tl
        oc[slot, 0:32, :] = cl
        pltpu.make_async_copy(
            ot.at[slot, pl.ds(0, 32)], to_hbm.at[pl.ds(TAIL_LINE0, 32)],
            osem.at[slot, 0]).start()
        pltpu.make_async_copy(
            oc.at[slot, pl.ds(0, 32)], co_hbm.at[pl.ds(TAIL_LINE0, 32)],
            osem.at[slot, 1]).start()
        pltpu.make_async_copy(
            ot.at[slot, pl.ds(0, 32)], to_hbm.at[pl.ds(TAIL_LINE0, 32)],
            osem.at[slot, 0]).wait()
        pltpu.make_async_copy(
            oc.at[slot, pl.ds(0, 32)], co_hbm.at[pl.ds(TAIL_LINE0, 32)],
            osem.at[slot, 1]).wait()
        pltpu.make_async_copy(
            ot.at[1 - slot], to_hbm.at[pl.ds(0, HLINES)],
            osem.at[1 - slot, 0]).wait()
        pltpu.make_async_copy(
            oc.at[1 - slot], co_hbm.at[pl.ds(0, HLINES)],
            osem.at[1 - slot, 1]).wait()


def _to_lines(temb_t, cemb_t, tail_t, tail_c):
    return pl.pallas_call(
        _transpose_body,
        grid=(TGRID,),
        in_specs=[
            pl.BlockSpec(memory_space=pl.ANY),
            pl.BlockSpec(memory_space=pl.ANY),
            pl.BlockSpec((TAIL, DIM), lambda i: (0, 0)),
            pl.BlockSpec((TAIL, DIM), lambda i: (0, 0)),
        ],
        out_specs=[
            pl.BlockSpec(memory_space=pl.ANY),
            pl.BlockSpec(memory_space=pl.ANY),
        ],
        out_shape=[
            jax.ShapeDtypeStruct((LINES, LINE_W), jnp.float32),
            jax.ShapeDtypeStruct((LINES, LINE_W), jnp.float32),
        ],
        scratch_shapes=[
            pltpu.VMEM((2, 2, DIM, W), jnp.float32),
            pltpu.VMEM((2, HLINES, LINE_W), jnp.float32),
            pltpu.VMEM((2, HLINES, LINE_W), jnp.float32),
            pltpu.SemaphoreType.DMA((2, 2)),
            pltpu.SemaphoreType.DMA((2, 2)),
        ],
    )(temb_t, cemb_t, tail_t, tail_c)


def _body(tw_hbm, cw_hbm, temb_hbm, cemb_hbm, out_hbm,
          idx_t, idx_c, line_t, line_c,
          rt0, rt1, rc0, rc1, out_v,
          st0, st1, sc0, sc1):
    cid = lax.axis_index("c")
    sid = lax.axis_index("s")
    wid = sid * NC + cid
    base = wid * BPW

    def to_line(v):
        main = ((v >> 9) << 8) + (v & 255)
        return jnp.where(v >= MAIN, TAIL_LINE0 + (v & 31), main)

    for j in range(NCHUNK):
        pltpu.sync_copy(tw_hbm.at[pl.ds(base + j * CH, CH)], idx_t.at[j])
        pltpu.sync_copy(cw_hbm.at[pl.ds(base + j * CH, CH)], idx_c.at[j])
    for j in range(NCHUNK):
        for k in range(CH // L):
            line_t[j, pl.ds(k * L, L)] = to_line(idx_t[j, pl.ds(k * L, L)])
            line_c[j, pl.ds(k * L, L)] = to_line(idx_c[j, pl.ds(k * L, L)])

    def copy_t(ch, buf, sem):
        return pltpu.make_async_copy(temb_hbm.at[line_t.at[ch]], buf, sem)

    def copy_c(ch, buf, sem):
        return pltpu.make_async_copy(cemb_hbm.at[line_c.at[ch]], buf, sem)

    copy_t(0, rt0, st0).start()
    copy_c(0, rc0, sc0).start()
    copy_t(1, rt1, st1).start()
    copy_c(1, rc1, sc1).start()

    lane = lax.iota(jnp.int32, L)

    def compute(ch, rt, rc):
        for g in range(CH // L):
            lidx = g * L + lane
            vt = idx_t[ch, pl.ds(g * L, L)]
            vc = idx_c[ch, pl.ds(g * L, L)]
            half_t = jnp.where(vt >= MAIN, (vt >> 5) & 1, (vt >> 8) & 1) * DIM
            half_c = jnp.where(vc >= MAIN, (vc >> 5) & 1, (vc >> 8) & 1) * DIM
            acc = jnp.zeros((L,), jnp.float32)
            for d in range(DIM):
                tv = plsc.load_gather(rt, [lidx, half_t + d])
                cv = plsc.load_gather(rc, [lidx, half_c + d])
                acc = acc + tv * cv
            out_v[pl.ds(ch * CH + g * L, L)] = acc

    def step(s, carry):
        ch0 = 2 * s
        ch1 = 2 * s + 1
        copy_t(ch0, rt0, st0).wait()
        copy_c(ch0, rc0, sc0).wait()
        compute(ch0, rt0, rc0)
        @pl.when(ch0 + 2 < NCHUNK)
        def _():
            copy_t(ch0 + 2, rt0, st0).start()
            copy_c(ch0 + 2, rc0, sc0).start()
        copy_t(ch1, rt1, st1).wait()
        copy_c(ch1, rc1, sc1).wait()
        compute(ch1, rt1, rc1)
        @pl.when(ch1 + 2 < NCHUNK)
        def _():
            copy_t(ch1 + 2, rt1, st1).start()
            copy_c(ch1 + 2, rc1, sc1).start()
        return carry

    lax.fori_loop(0, NCHUNK // 2, step, 0)

    pltpu.sync_copy(out_v, out_hbm.at[pl.ds(base, BPW)])


def kernel(target_word, context_word, target_emb, context_emb):
    tw = target_word.astype(jnp.int32)
    cw = context_word.astype(jnp.int32)
    temb_d, cemb_d = _to_lines(
        target_emb.T, context_emb.T,
        target_emb[MAIN:], context_emb[MAIN:])
    mesh = plsc.VectorSubcoreMesh(
        core_axis_name="c", subcore_axis_name="s",
        num_cores=NC, num_subcores=NS)
    run = pl.kernel(
        _body,
        out_type=jax.ShapeDtypeStruct((BATCH,), jnp.float32),
        mesh=mesh,
        scratch_types=[
            pltpu.VMEM((NCHUNK, CH), jnp.int32),
            pltpu.VMEM((NCHUNK, CH), jnp.int32),
            pltpu.VMEM((NCHUNK, CH), jnp.int32),
            pltpu.VMEM((NCHUNK, CH), jnp.int32),
            pltpu.VMEM((CH, LINE_W), jnp.float32),
            pltpu.VMEM((CH, LINE_W), jnp.float32),
            pltpu.VMEM((CH, LINE_W), jnp.float32),
            pltpu.VMEM((CH, LINE_W), jnp.float32),
            pltpu.VMEM((BPW,), jnp.float32),
            pltpu.SemaphoreType.DMA,
            pltpu.SemaphoreType.DMA,
            pltpu.SemaphoreType.DMA,
            pltpu.SemaphoreType.DMA,
        ],
        compiler_params=pltpu.CompilerParams(
            needs_layout_passes=False, use_tc_tiling_on_sc=True),
    )
    return run(tw, cw, temb_d, cemb_d)
